# baseline (speedup 1.0000x reference)
import functools

import jax
import jax.numpy as jnp
from jax import lax
from jax.experimental import pallas as pl
from jax.experimental.pallas import tpu as pltpu
from jax.experimental.pallas import tpu_sc as plsc

N_ATOMS = 1600000
N_SEG = 50000

NC = 2
NS = 16
NW = NC * NS

PER_TILE = N_ATOMS // NW
N_CHUNK = 5
CHUNK = PER_TILE // N_CHUNK
LSTRIDE = CHUNK // 16
ACC = 50048


def _sc_body(e_hbm, b_hbm, out_hbm, accum, ib0, vb0, ib1, vb1, s0, s1, s2, s3):
    c = lax.axis_index("c")
    s = lax.axis_index("s")
    wid = s * NC + c
    base = wid * PER_TILE

    ibufs = (ib0, ib1)
    vbufs = (vb0, vb1)
    bsems = (s0, s1)
    vsems = (s2, s3)

    def start(k, slot):
        off = base + k * CHUNK
        cb = pltpu.async_copy(b_hbm.at[pl.ds(off, CHUNK)], ibufs[slot], bsems[slot])
        ce = pltpu.async_copy(e_hbm.at[pl.ds(off, CHUNK)], vbufs[slot], vsems[slot])
        return (cb, ce)

    descs = [None, None]
    descs[0] = start(0, 0)

    def _zero(i, carry):
        for u in range(8):
            accum[pl.ds((i * 8 + u) * 16, 16)] = jnp.zeros((16,), jnp.float32)
        return carry

    lax.fori_loop(0, ACC // 128, _zero, 0)

    lane_off = lax.broadcasted_iota(jnp.int32, (16,), 0) * LSTRIDE

    for k in range(N_CHUNK):
        slot = k % 2
        if k + 1 < N_CHUNK:
            descs[(k + 1) % 2] = start(k + 1, (k + 1) % 2)
        descs[slot][0].wait()
        descs[slot][1].wait()
        ib = ibufs[slot]
        vb = vbufs[slot]

        @plsc.parallel_loop(0, LSTRIDE, unroll=25)
        def _acc(i):
            off = lane_off + i
            bvec = plsc.load_gather(ib, [off])
            evec = plsc.load_gather(vb, [off])
            plsc.addupdate_scatter(accum, [bvec], evec)

    pltpu.sync_copy(accum, out_hbm.at[wid])


_sc_kernel = functools.partial(
    pl.kernel,
    out_type=jax.ShapeDtypeStruct((NW, ACC), jnp.float32),
    mesh=plsc.VectorSubcoreMesh(
        core_axis_name="c", subcore_axis_name="s", num_cores=NC, num_subcores=NS
    ),
    scratch_types=[
        pltpu.VMEM((ACC,), jnp.float32),
        pltpu.VMEM((CHUNK,), jnp.int32),
        pltpu.VMEM((CHUNK,), jnp.float32),
        pltpu.VMEM((CHUNK,), jnp.int32),
        pltpu.VMEM((CHUNK,), jnp.float32),
        pltpu.SemaphoreType.DMA,
        pltpu.SemaphoreType.DMA,
        pltpu.SemaphoreType.DMA,
        pltpu.SemaphoreType.DMA,
    ],
    compiler_params=pltpu.CompilerParams(needs_layout_passes=False),
)(_sc_body)


def _tc_reduce(x_ref, o_ref):
    o_ref[...] = jnp.sum(x_ref[...], axis=0)


@jax.jit
def kernel(energy, batch):
    partials = _sc_kernel(energy, batch)
    out = pl.pallas_call(
        _tc_reduce,
        out_shape=jax.ShapeDtypeStruct((ACC // 128, 128), jnp.float32),
    )(partials.reshape(NW, ACC // 128, 128))
    return out.reshape(ACC)[:N_SEG]


# speedup vs baseline: 1.0320x; 1.0320x over previous
import functools

import jax
import jax.numpy as jnp
from jax import lax
from jax.experimental import pallas as pl
from jax.experimental.pallas import tpu as pltpu
from jax.experimental.pallas import tpu_sc as plsc

N_ATOMS = 1600000
N_SEG = 50000

NC = 2
NS = 16
NW = NC * NS

PER_TILE = N_ATOMS // NW
N_CHUNK = 5
CHUNK = PER_TILE // N_CHUNK
LSTRIDE = CHUNK // 16
ACC = 50048


def _sc_body(e_hbm, b_hbm, out_hbm, accum, ib0, vb0, ib1, vb1, s0, s1, s2, s3):
    c = lax.axis_index("c")
    s = lax.axis_index("s")
    wid = s * NC + c
    base = wid * PER_TILE

    ibufs = (ib0, ib1)
    vbufs = (vb0, vb1)
    bsems = (s0, s1)
    vsems = (s2, s3)

    def start(k, slot):
        off = base + k * CHUNK
        cb = pltpu.async_copy(b_hbm.at[pl.ds(off, CHUNK)], ibufs[slot], bsems[slot])
        ce = pltpu.async_copy(e_hbm.at[pl.ds(off, CHUNK)], vbufs[slot], vsems[slot])
        return (cb, ce)

    descs = [None, None]
    descs[0] = start(0, 0)

    def _zero(i, carry):
        for u in range(8):
            accum[pl.ds((i * 8 + u) * 16, 16)] = jnp.zeros((16,), jnp.float32)
        return carry

    lax.fori_loop(0, ACC // 128, _zero, 0)

    lane_off = lax.broadcasted_iota(jnp.int32, (16,), 0) * LSTRIDE

    for k in range(N_CHUNK):
        slot = k % 2
        if k + 1 < N_CHUNK:
            descs[(k + 1) % 2] = start(k + 1, (k + 1) % 2)
        descs[slot][0].wait()
        descs[slot][1].wait()
        ib = ibufs[slot]
        vb = vbufs[slot]

        @plsc.parallel_loop(0, LSTRIDE, unroll=5)
        def _acc(i):
            off = lane_off + i
            bvec = plsc.load_gather(ib, [off])
            evec = plsc.load_gather(vb, [off])
            plsc.addupdate_scatter(accum, [bvec], evec)

    pltpu.sync_copy(accum, out_hbm.at[wid])


_sc_kernel = functools.partial(
    pl.kernel,
    out_type=jax.ShapeDtypeStruct((NW, ACC), jnp.float32),
    mesh=plsc.VectorSubcoreMesh(
        core_axis_name="c", subcore_axis_name="s", num_cores=NC, num_subcores=NS
    ),
    scratch_types=[
        pltpu.VMEM((ACC,), jnp.float32),
        pltpu.VMEM((CHUNK,), jnp.int32),
        pltpu.VMEM((CHUNK,), jnp.float32),
        pltpu.VMEM((CHUNK,), jnp.int32),
        pltpu.VMEM((CHUNK,), jnp.float32),
        pltpu.SemaphoreType.DMA,
        pltpu.SemaphoreType.DMA,
        pltpu.SemaphoreType.DMA,
        pltpu.SemaphoreType.DMA,
    ],
    compiler_params=pltpu.CompilerParams(needs_layout_passes=False),
)(_sc_body)


def _tc_reduce(x_ref, o_ref):
    o_ref[...] = jnp.sum(x_ref[...], axis=0)


@jax.jit
def kernel(energy, batch):
    partials = _sc_kernel(energy, batch)
    out = pl.pallas_call(
        _tc_reduce,
        out_shape=jax.ShapeDtypeStruct((ACC // 128, 128), jnp.float32),
    )(partials.reshape(NW, ACC // 128, 128))
    return out.reshape(ACC)[:N_SEG]


# speedup vs baseline: 1.2181x; 1.1803x over previous
import functools

import jax
import jax.numpy as jnp
from jax import lax
from jax.experimental import pallas as pl
from jax.experimental.pallas import tpu as pltpu
from jax.experimental.pallas import tpu_sc as plsc

N_ATOMS = 1600000
N_SEG = 50000

NC = 2
NS = 16
NW = NC * NS

PER_TILE = N_ATOMS // NW
N_CHUNK = 5
CHUNK = PER_TILE // N_CHUNK
LSTRIDE = CHUNK // 16
ACC = 50048


def _sc_body(e_hbm, b_hbm, out_hbm, accum, ib0, vb0, ib1, vb1, s0, s1, s2, s3):
    c = lax.axis_index("c")
    s = lax.axis_index("s")
    wid = s * NC + c
    base = wid * PER_TILE

    ibufs = (ib0, ib1)
    vbufs = (vb0, vb1)
    bsems = (s0, s1)
    vsems = (s2, s3)

    def start(k, slot):
        off = base + k * CHUNK
        cb = pltpu.async_copy(b_hbm.at[pl.ds(off, CHUNK)], ibufs[slot], bsems[slot])
        ce = pltpu.async_copy(e_hbm.at[pl.ds(off, CHUNK)], vbufs[slot], vsems[slot])
        return (cb, ce)

    descs = [None, None]
    descs[0] = start(0, 0)

    def _zero(i, carry):
        for u in range(8):
            accum[pl.ds((i * 8 + u) * 16, 16)] = jnp.zeros((16,), jnp.float32)
        return carry

    lax.fori_loop(0, ACC // 128, _zero, 0)

    lane_off = lax.broadcasted_iota(jnp.int32, (16,), 0) * LSTRIDE

    for k in range(N_CHUNK):
        slot = k % 2
        if k + 1 < N_CHUNK:
            descs[(k + 1) % 2] = start(k + 1, (k + 1) % 2)
        descs[slot][0].wait()
        descs[slot][1].wait()
        ib = ibufs[slot]
        vb = vbufs[slot]

        @plsc.parallel_loop(0, LSTRIDE, unroll=5)
        def _acc(i):
            off = lane_off + i
            bvec = plsc.load_gather(ib, [off])
            evec = plsc.load_gather(vb, [off])
            plsc.addupdate_scatter(accum, [bvec], evec)

    pltpu.sync_copy(accum, out_hbm.at[wid])


_sc_kernel = functools.partial(
    pl.kernel,
    out_type=jax.ShapeDtypeStruct((NW, ACC), jnp.float32),
    mesh=plsc.VectorSubcoreMesh(
        core_axis_name="c", subcore_axis_name="s", num_cores=NC, num_subcores=NS
    ),
    scratch_types=[
        pltpu.VMEM((ACC,), jnp.float32),
        pltpu.VMEM((CHUNK,), jnp.int32),
        pltpu.VMEM((CHUNK,), jnp.float32),
        pltpu.VMEM((CHUNK,), jnp.int32),
        pltpu.VMEM((CHUNK,), jnp.float32),
        pltpu.SemaphoreType.DMA,
        pltpu.SemaphoreType.DMA,
        pltpu.SemaphoreType.DMA,
        pltpu.SemaphoreType.DMA,
    ],
    compiler_params=pltpu.CompilerParams(needs_layout_passes=False),
)(_sc_body)


def _tc_reduce(x_ref, o_ref):
    o_ref[...] = jnp.sum(x_ref[...], axis=0)


@jax.jit
def kernel(energy, batch):
    partials = _sc_kernel(energy, batch)
    out = pl.pallas_call(
        _tc_reduce,
        out_shape=jax.ShapeDtypeStruct((ACC,), jnp.float32),
    )(partials)
    return out[:N_SEG]
